# Initial kernel scaffold; baseline (speedup 1.0000x reference)
import jax
import jax.numpy as jnp
from jax.experimental import pallas as pl


def kernel(hidden_states, routing_indices, routing_weights):
    raise NotImplementedError("write your pallas kernel here")


# speedup vs baseline: 28.0301x; 28.0301x over previous
import functools

import jax
import jax.numpy as jnp
from jax.experimental import pallas as pl
from jax.experimental.pallas import tpu as pltpu


def _rowscale_kernel(h_ref, w_ref, o_ref):
    h = h_ref[...]
    w = w_ref[...]
    topk = w.shape[1]
    acc = h * w[:, 0:1]
    for k in range(1, topk):
        acc = acc + h * w[:, k : k + 1]
    o_ref[...] = acc


@functools.partial(jax.jit, static_argnames=())
def kernel(hidden_states, routing_indices, routing_weights):
    del routing_indices
    num_tokens, hidden_dim = hidden_states.shape
    topk = routing_weights.shape[1]
    w = routing_weights.astype(hidden_states.dtype)

    blk = 512
    while num_tokens % blk != 0:
        blk //= 2
    grid = (num_tokens // blk,)

    return pl.pallas_call(
        _rowscale_kernel,
        grid=grid,
        in_specs=[
            pl.BlockSpec((blk, hidden_dim), lambda i: (i, 0)),
            pl.BlockSpec((blk, topk), lambda i: (i, 0)),
        ],
        out_specs=pl.BlockSpec((blk, hidden_dim), lambda i: (i, 0)),
        out_shape=jax.ShapeDtypeStruct((num_tokens, hidden_dim), hidden_states.dtype),
        compiler_params=pltpu.CompilerParams(
            dimension_semantics=("arbitrary",),
        ),
    )(hidden_states, w)


# speedup vs baseline: 28.6072x; 1.0206x over previous
import functools

import jax
import jax.numpy as jnp
from jax.experimental import pallas as pl
from jax.experimental.pallas import tpu as pltpu


def _rowscale_kernel(h_ref, w_ref, o_ref):
    h = h_ref[...]
    w = w_ref[...]
    topk = w.shape[1]
    acc = h * w[:, 0:1]
    for k in range(1, topk):
        acc = acc + h * w[:, k : k + 1]
    o_ref[...] = acc


@functools.partial(jax.jit, static_argnames=())
def kernel(hidden_states, routing_indices, routing_weights):
    del routing_indices
    num_tokens, hidden_dim = hidden_states.shape
    topk = routing_weights.shape[1]
    w = routing_weights.astype(hidden_states.dtype)

    blk = 1024
    while num_tokens % blk != 0:
        blk //= 2
    grid = (num_tokens // blk,)

    return pl.pallas_call(
        _rowscale_kernel,
        grid=grid,
        in_specs=[
            pl.BlockSpec((blk, hidden_dim), lambda i: (i, 0)),
            pl.BlockSpec((blk, topk), lambda i: (i, 0)),
        ],
        out_specs=pl.BlockSpec((blk, hidden_dim), lambda i: (i, 0)),
        out_shape=jax.ShapeDtypeStruct((num_tokens, hidden_dim), hidden_states.dtype),
        compiler_params=pltpu.CompilerParams(
            dimension_semantics=("arbitrary",),
        ),
    )(hidden_states, w)
